# baseline (speedup 1.0000x reference)
import jax
import jax.numpy as jnp
import numpy as np
from jax import lax
from jax.experimental import pallas as pl
from jax.experimental.pallas import tpu as pltpu
from jax.experimental.pallas import tpu_sc as plsc

NC, NS, L = 2, 16, 16
NW = NC * NS

VOCAB = 100000
BATCH, SEQ = 4096, 200
EMBED, HIDDEN = 128, 128
EMBED_W = EMBED // 2
SEQ_HALF = SEQ // 2
ROWS_PER_W = BATCH // NW
NCHUNK = EMBED // L

VROWS_PER_W = VOCAB // NW
VBLK = 125


def _pack16(x32f_lo, x32f_hi):
    def rte(v):
        u = lax.bitcast_convert_type(v, jnp.int32)
        bias = 0x7FFF + jnp.bitwise_and(lax.shift_right_logical(u, 16), 1)
        return lax.shift_right_logical(u + bias, 16)
    return jnp.bitwise_or(rte(x32f_lo),
                          lax.shift_left(rte(x32f_hi), 16))


def _conv_body(table_hbm, out_hbm, in_v, out_v, isem0, isem1, osem0, osem1):
    wid = lax.axis_index("s") * NC + lax.axis_index("c")
    base = wid * VROWS_PER_W
    isems = (isem0, isem1)
    osems = (osem0, osem1)

    def issue(slot, t):
        pltpu.async_copy(table_hbm.at[pl.ds(base + VBLK * t, VBLK)],
                         in_v.at[slot], isems[slot])

    def wait_in(slot, t):
        pltpu.make_async_copy(table_hbm.at[pl.ds(base + VBLK * t, VBLK)],
                              in_v.at[slot], isems[slot]).wait()

    issue(0, 0)

    def block_step(t2, carry):
        for sub in range(2):
            t = 2 * t2 + sub
            slot = sub

            @pl.when(t + 1 < VROWS_PER_W // VBLK)
            def _():
                issue(1 - slot, t + 1)

            wait_in(slot, t)

            def row_step(r, c2):
                for c in range(EMBED_W // L):
                    lo = in_v[slot, r, pl.ds(L * c, L)]
                    hi = in_v[slot, r, pl.ds(EMBED_W + L * c, L)]
                    out_v[slot, r, pl.ds(L * c, L)] = _pack16(lo, hi)
                return c2

            lax.fori_loop(0, VBLK, row_step, 0)

            @pl.when(t2 > 0)
            def _():
                pltpu.make_async_copy(
                    out_v.at[slot],
                    out_hbm.at[pl.ds(base + VBLK * t, VBLK)],
                    osems[slot]).wait()

            pltpu.async_copy(out_v.at[slot],
                             out_hbm.at[pl.ds(base + VBLK * t, VBLK)],
                             osems[slot])
        return carry

    nblk = VROWS_PER_W // VBLK
    lax.fori_loop(0, nblk // 2, block_step, 0)
    t = nblk - 1
    wait_in(0, t)

    def row_step_t(r, c2):
        for c in range(EMBED_W // L):
            lo = in_v[0, r, pl.ds(L * c, L)]
            hi = in_v[0, r, pl.ds(EMBED_W + L * c, L)]
            out_v[0, r, pl.ds(L * c, L)] = _pack16(lo, hi)
        return c2

    lax.fori_loop(0, VBLK, row_step_t, 0)
    pltpu.make_async_copy(out_v.at[0],
                          out_hbm.at[pl.ds(base + VBLK * (t - 2), VBLK)],
                          osems[0]).wait()
    pltpu.make_async_copy(out_v.at[1],
                          out_hbm.at[pl.ds(base + VBLK * (t - 1), VBLK)],
                          osems[1]).wait()
    pltpu.async_copy(out_v.at[0], out_hbm.at[pl.ds(base + VBLK * t, VBLK)],
                     osems[0])
    pltpu.make_async_copy(out_v.at[0],
                          out_hbm.at[pl.ds(base + VBLK * t, VBLK)],
                          osems[0]).wait()


_conv = pl.kernel(
    _conv_body,
    out_type=jax.ShapeDtypeStruct((VOCAB, EMBED_W), jnp.int32),
    mesh=plsc.VectorSubcoreMesh(core_axis_name="c", subcore_axis_name="s",
                                num_cores=NC, num_subcores=NS),
    compiler_params=pltpu.CompilerParams(use_tc_tiling_on_sc=False),
    scratch_types=[
        pltpu.VMEM((2, VBLK, EMBED), jnp.float32),
        pltpu.VMEM((2, VBLK, EMBED_W), jnp.int32),
        pltpu.SemaphoreType.DMA,
        pltpu.SemaphoreType.DMA,
        pltpu.SemaphoreType.DMA,
        pltpu.SemaphoreType.DMA,
    ],
)


def _pool_body(ids_hbm, table_hbm, out_hbm, idx_all, rows_v, acc_v,
               s00, s01, s10, s11, osem0, osem1):
    wid = lax.axis_index("s") * NC + lax.axis_index("c")
    base = wid * ROWS_PER_W
    sems = ((s00, s01), (s10, s11))
    osems = (osem0, osem1)

    pltpu.sync_copy(ids_hbm.at[pl.ds(base, ROWS_PER_W)], idx_all)

    def issue(slot, r):
        for j in range(2):
            pltpu.async_copy(table_hbm.at[idx_all.at[r, j]],
                             rows_v.at[slot, j], sems[slot][j])

    def wait(slot, r, j):
        pltpu.make_async_copy(table_hbm.at[idx_all.at[r, j]],
                              rows_v.at[slot, j], sems[slot][j]).wait()

    def chunk_reduce(slot, j, acc0):
        def seq_step(s, acc):
            acc = list(acc)
            for k2 in range(NCHUNK // 2):
                x32 = rows_v[slot, j, s, pl.ds(L * k2, L)]
                a = lax.bitcast_convert_type(
                    lax.shift_left(x32, 16), jnp.float32)
                bb = lax.bitcast_convert_type(
                    jnp.bitwise_and(x32, jnp.int32(-65536)), jnp.float32)
                acc[2 * k2] = acc[2 * k2] + a
                acc[2 * k2 + 1] = acc[2 * k2 + 1] + bb
            return tuple(acc)

        return lax.fori_loop(0, SEQ_HALF, seq_step, acc0)

    def reduce_store(slot, r, i):
        wait(slot, r, 0)
        acc = chunk_reduce(
            slot, 0, tuple(jnp.zeros((L,), jnp.float32)
                           for _ in range(NCHUNK)))
        wait(slot, r, 1)
        acc = chunk_reduce(slot, 1, acc)

        @pl.when(i > 0)
        def _():
            pltpu.make_async_copy(acc_v.at[slot], out_hbm.at[base + r],
                                  osems[slot]).wait()

        for k in range(NCHUNK):
            acc_v[slot, pl.ds(L * k, L)] = acc[k] * (1.0 / SEQ)
        pltpu.async_copy(acc_v.at[slot], out_hbm.at[base + r], osems[slot])

    issue(0, 0)

    def pair_step(i, carry):
        r0 = 2 * i
        issue(1, r0 + 1)
        reduce_store(0, r0, i)

        @pl.when(r0 + 2 < ROWS_PER_W)
        def _():
            issue(0, r0 + 2)

        reduce_store(1, r0 + 1, i)
        return carry

    lax.fori_loop(0, ROWS_PER_W // 2, pair_step, 0)

    last = ROWS_PER_W - 2
    for slot in range(2):
        pltpu.make_async_copy(acc_v.at[slot], out_hbm.at[base + last + slot],
                              osems[slot]).wait()


_pool = pl.kernel(
    _pool_body,
    out_type=jax.ShapeDtypeStruct((BATCH, EMBED), jnp.float32),
    mesh=plsc.VectorSubcoreMesh(core_axis_name="c", subcore_axis_name="s",
                                num_cores=NC, num_subcores=NS),
    compiler_params=pltpu.CompilerParams(use_tc_tiling_on_sc=False),
    scratch_types=[
        pltpu.VMEM((ROWS_PER_W, 2, SEQ_HALF), jnp.int32),
        pltpu.VMEM((2, 2, SEQ_HALF, EMBED_W), jnp.int32),
        pltpu.VMEM((2, EMBED), jnp.float32),
        pltpu.SemaphoreType.DMA,
        pltpu.SemaphoreType.DMA,
        pltpu.SemaphoreType.DMA,
        pltpu.SemaphoreType.DMA,
        pltpu.SemaphoreType.DMA,
        pltpu.SemaphoreType.DMA,
    ],
)


def _linear_body(p_ref, w_ref, b_ref, o_ref):
    o_ref[...] = lax.dot_general(
        p_ref[...], w_ref[...], (((1,), (1,)), ((), ())),
        preferred_element_type=jnp.float32) + b_ref[...]


_PERM = np.concatenate(
    [np.concatenate([16 * k2 + np.arange(16),
                     64 + 16 * k2 + np.arange(16)])
     for k2 in range(4)]).astype(np.int32)


def kernel(input_ids, table, W, b):
    ids2 = input_ids.astype(jnp.int32).reshape(BATCH, 2, SEQ_HALF)
    packed = _conv(table)
    pooled = _pool(ids2, packed)
    out = pl.pallas_call(
        _linear_body,
        out_shape=jax.ShapeDtypeStruct((BATCH, HIDDEN), jnp.float32),
        grid=(BATCH // 1024,),
        in_specs=[
            pl.BlockSpec((1024, EMBED), lambda i: (i, 0)),
            pl.BlockSpec((HIDDEN, EMBED), lambda i: (0, 0)),
            pl.BlockSpec((1, HIDDEN), lambda i: (0, 0)),
        ],
        out_specs=pl.BlockSpec((1024, HIDDEN), lambda i: (i, 0)),
    )(pooled, W[:, _PERM], b.reshape(1, HIDDEN))
    return out


# speedup vs baseline: 1.0958x; 1.0958x over previous
import jax
import jax.numpy as jnp
import numpy as np
from jax import lax
from jax.experimental import pallas as pl
from jax.experimental.pallas import tpu as pltpu
from jax.experimental.pallas import tpu_sc as plsc

NC, NS, L = 2, 16, 16
NW = NC * NS

VOCAB = 100000
BATCH, SEQ = 4096, 200
EMBED, HIDDEN = 128, 128
EMBED_W = EMBED // 2
SEQ_HALF = SEQ // 2
ROWS_PER_W = BATCH // NW
NCHUNK = EMBED // L

VROWS_PER_W = VOCAB // NW
VBLK = 125


def _pack16(x32f_lo, x32f_hi):
    ulo = lax.bitcast_convert_type(x32f_lo, jnp.int32)
    uhi = lax.bitcast_convert_type(x32f_hi, jnp.int32)
    lo16 = lax.shift_right_logical(ulo + 0x8000, 16)
    hi16 = jnp.bitwise_and(uhi + 0x8000, jnp.int32(-65536))
    return jnp.bitwise_or(lo16, hi16)


def _conv_body(table_hbm, out_hbm, in_v, out_v, isem0, isem1, osem0, osem1):
    wid = lax.axis_index("s") * NC + lax.axis_index("c")
    base = wid * VROWS_PER_W
    isems = (isem0, isem1)
    osems = (osem0, osem1)

    def issue(slot, t):
        pltpu.async_copy(table_hbm.at[pl.ds(base + VBLK * t, VBLK)],
                         in_v.at[slot], isems[slot])

    def wait_in(slot, t):
        pltpu.make_async_copy(table_hbm.at[pl.ds(base + VBLK * t, VBLK)],
                              in_v.at[slot], isems[slot]).wait()

    issue(0, 0)

    def block_step(t2, carry):
        for sub in range(2):
            t = 2 * t2 + sub
            slot = sub

            @pl.when(t + 1 < VROWS_PER_W // VBLK)
            def _():
                issue(1 - slot, t + 1)

            wait_in(slot, t)

            def row_step(r, c2):
                for c in range(EMBED_W // L):
                    lo = in_v[slot, r, pl.ds(L * c, L)]
                    hi = in_v[slot, r, pl.ds(EMBED_W + L * c, L)]
                    out_v[slot, r, pl.ds(L * c, L)] = _pack16(lo, hi)
                return c2

            lax.fori_loop(0, VBLK, row_step, 0)

            @pl.when(t2 > 0)
            def _():
                pltpu.make_async_copy(
                    out_v.at[slot],
                    out_hbm.at[pl.ds(base + VBLK * t, VBLK)],
                    osems[slot]).wait()

            pltpu.async_copy(out_v.at[slot],
                             out_hbm.at[pl.ds(base + VBLK * t, VBLK)],
                             osems[slot])
        return carry

    nblk = VROWS_PER_W // VBLK
    lax.fori_loop(0, nblk // 2, block_step, 0)
    t = nblk - 1
    wait_in(0, t)

    def row_step_t(r, c2):
        for c in range(EMBED_W // L):
            lo = in_v[0, r, pl.ds(L * c, L)]
            hi = in_v[0, r, pl.ds(EMBED_W + L * c, L)]
            out_v[0, r, pl.ds(L * c, L)] = _pack16(lo, hi)
        return c2

    lax.fori_loop(0, VBLK, row_step_t, 0)
    pltpu.make_async_copy(out_v.at[0],
                          out_hbm.at[pl.ds(base + VBLK * (t - 2), VBLK)],
                          osems[0]).wait()
    pltpu.make_async_copy(out_v.at[1],
                          out_hbm.at[pl.ds(base + VBLK * (t - 1), VBLK)],
                          osems[1]).wait()
    pltpu.async_copy(out_v.at[0], out_hbm.at[pl.ds(base + VBLK * t, VBLK)],
                     osems[0])
    pltpu.make_async_copy(out_v.at[0],
                          out_hbm.at[pl.ds(base + VBLK * t, VBLK)],
                          osems[0]).wait()


_conv = pl.kernel(
    _conv_body,
    out_type=jax.ShapeDtypeStruct((VOCAB, EMBED_W), jnp.int32),
    mesh=plsc.VectorSubcoreMesh(core_axis_name="c", subcore_axis_name="s",
                                num_cores=NC, num_subcores=NS),
    compiler_params=pltpu.CompilerParams(use_tc_tiling_on_sc=False),
    scratch_types=[
        pltpu.VMEM((2, VBLK, EMBED), jnp.float32),
        pltpu.VMEM((2, VBLK, EMBED_W), jnp.int32),
        pltpu.SemaphoreType.DMA,
        pltpu.SemaphoreType.DMA,
        pltpu.SemaphoreType.DMA,
        pltpu.SemaphoreType.DMA,
    ],
)


def _pool_body(ids_hbm, table_hbm, out_hbm, idx_all, rows_v, acc_v,
               s00, s01, s10, s11, osem0, osem1):
    wid = lax.axis_index("s") * NC + lax.axis_index("c")
    base = wid * ROWS_PER_W
    sems = ((s00, s01), (s10, s11))
    osems = (osem0, osem1)

    pltpu.sync_copy(ids_hbm.at[pl.ds(base, ROWS_PER_W)], idx_all)

    def issue(slot, r):
        for j in range(2):
            pltpu.async_copy(table_hbm.at[idx_all.at[r, j]],
                             rows_v.at[slot, j], sems[slot][j])

    def wait(slot, r, j):
        pltpu.make_async_copy(table_hbm.at[idx_all.at[r, j]],
                              rows_v.at[slot, j], sems[slot][j]).wait()

    def chunk_reduce(slot, j, acc0):
        def seq_step(s, acc):
            acc = list(acc)
            for u in range(2):
                for k2 in range(NCHUNK // 2):
                    x32 = rows_v[slot, j, 2 * s + u, pl.ds(L * k2, L)]
                    a = lax.bitcast_convert_type(
                        lax.shift_left(x32, 16), jnp.float32)
                    bb = lax.bitcast_convert_type(
                        jnp.bitwise_and(x32, jnp.int32(-65536)),
                        jnp.float32)
                    acc[2 * k2] = acc[2 * k2] + a
                    acc[2 * k2 + 1] = acc[2 * k2 + 1] + bb
            return tuple(acc)

        return lax.fori_loop(0, SEQ_HALF // 2, seq_step, acc0)

    def reduce_store(slot, r, i):
        wait(slot, r, 0)
        acc = chunk_reduce(
            slot, 0, tuple(jnp.zeros((L,), jnp.float32)
                           for _ in range(NCHUNK)))
        wait(slot, r, 1)
        acc = chunk_reduce(slot, 1, acc)

        @pl.when(i > 0)
        def _():
            pltpu.make_async_copy(acc_v.at[slot], out_hbm.at[base + r],
                                  osems[slot]).wait()

        for k in range(NCHUNK):
            acc_v[slot, pl.ds(L * k, L)] = acc[k] * (1.0 / SEQ)
        pltpu.async_copy(acc_v.at[slot], out_hbm.at[base + r], osems[slot])

    issue(0, 0)

    def pair_step(i, carry):
        r0 = 2 * i
        issue(1, r0 + 1)
        reduce_store(0, r0, i)

        @pl.when(r0 + 2 < ROWS_PER_W)
        def _():
            issue(0, r0 + 2)

        reduce_store(1, r0 + 1, i)
        return carry

    lax.fori_loop(0, ROWS_PER_W // 2, pair_step, 0)

    last = ROWS_PER_W - 2
    for slot in range(2):
        pltpu.make_async_copy(acc_v.at[slot], out_hbm.at[base + last + slot],
                              osems[slot]).wait()


_pool = pl.kernel(
    _pool_body,
    out_type=jax.ShapeDtypeStruct((BATCH, EMBED), jnp.float32),
    mesh=plsc.VectorSubcoreMesh(core_axis_name="c", subcore_axis_name="s",
                                num_cores=NC, num_subcores=NS),
    compiler_params=pltpu.CompilerParams(use_tc_tiling_on_sc=False),
    scratch_types=[
        pltpu.VMEM((ROWS_PER_W, 2, SEQ_HALF), jnp.int32),
        pltpu.VMEM((2, 2, SEQ_HALF, EMBED_W), jnp.int32),
        pltpu.VMEM((2, EMBED), jnp.float32),
        pltpu.SemaphoreType.DMA,
        pltpu.SemaphoreType.DMA,
        pltpu.SemaphoreType.DMA,
        pltpu.SemaphoreType.DMA,
        pltpu.SemaphoreType.DMA,
        pltpu.SemaphoreType.DMA,
    ],
)


def _linear_body(p_ref, w_ref, b_ref, o_ref):
    o_ref[...] = lax.dot_general(
        p_ref[...], w_ref[...], (((1,), (1,)), ((), ())),
        preferred_element_type=jnp.float32) + b_ref[...]


_PERM = np.concatenate(
    [np.concatenate([16 * k2 + np.arange(16),
                     64 + 16 * k2 + np.arange(16)])
     for k2 in range(4)]).astype(np.int32)


def kernel(input_ids, table, W, b):
    ids2 = input_ids.astype(jnp.int32).reshape(BATCH, 2, SEQ_HALF)
    packed = _conv(table)
    pooled = _pool(ids2, packed)
    out = pl.pallas_call(
        _linear_body,
        out_shape=jax.ShapeDtypeStruct((BATCH, HIDDEN), jnp.float32),
        grid=(BATCH // 1024,),
        in_specs=[
            pl.BlockSpec((1024, EMBED), lambda i: (i, 0)),
            pl.BlockSpec((HIDDEN, EMBED), lambda i: (0, 0)),
            pl.BlockSpec((1, HIDDEN), lambda i: (0, 0)),
        ],
        out_specs=pl.BlockSpec((1024, HIDDEN), lambda i: (i, 0)),
    )(pooled, W[:, _PERM], b.reshape(1, HIDDEN))
    return out


# speedup vs baseline: 1.2203x; 1.1136x over previous
import jax
import jax.numpy as jnp
import numpy as np
from jax import lax
from jax.experimental import pallas as pl
from jax.experimental.pallas import tpu as pltpu
from jax.experimental.pallas import tpu_sc as plsc

NC, NS, L = 2, 16, 16
NW = NC * NS

VOCAB = 100000
BATCH, SEQ = 4096, 200
EMBED, HIDDEN = 128, 128
EMBED_W = EMBED // 2
SEQ_HALF = SEQ // 2
ROWS_PER_W = BATCH // NW
NCHUNK = EMBED // L

VROWS_PER_W = VOCAB // NW
VBLK = 125


def _pack16(x32f_lo, x32f_hi):
    ulo = lax.bitcast_convert_type(x32f_lo, jnp.int32)
    uhi = lax.bitcast_convert_type(x32f_hi, jnp.int32)
    lo16 = lax.shift_right_logical(ulo + 0x8000, 16)
    hi16 = jnp.bitwise_and(uhi + 0x8000, jnp.int32(-65536))
    return jnp.bitwise_or(lo16, hi16)


def _conv_body(table_hbm, out_hbm, in_v, out_v, isem0, isem1, osem0, osem1):
    wid = lax.axis_index("s") * NC + lax.axis_index("c")
    base = wid * VROWS_PER_W
    isems = (isem0, isem1)
    osems = (osem0, osem1)

    def issue(slot, t):
        pltpu.async_copy(table_hbm.at[pl.ds(base + VBLK * t, VBLK)],
                         in_v.at[slot], isems[slot])

    def wait_in(slot, t):
        pltpu.make_async_copy(table_hbm.at[pl.ds(base + VBLK * t, VBLK)],
                              in_v.at[slot], isems[slot]).wait()

    issue(0, 0)

    def block_step(t2, carry):
        for sub in range(2):
            t = 2 * t2 + sub
            slot = sub

            @pl.when(t + 1 < VROWS_PER_W // VBLK)
            def _():
                issue(1 - slot, t + 1)

            wait_in(slot, t)

            def row_step(r5, c2):
                for rr in range(5):
                    r = 5 * r5 + rr
                    for c in range(EMBED_W // L):
                        lo = in_v[slot, r, pl.ds(L * c, L)]
                        hi = in_v[slot, r, pl.ds(EMBED_W + L * c, L)]
                        out_v[slot, r, pl.ds(L * c, L)] = _pack16(lo, hi)
                return c2

            lax.fori_loop(0, VBLK // 5, row_step, 0)

            @pl.when(t2 > 0)
            def _():
                pltpu.make_async_copy(
                    out_v.at[slot],
                    out_hbm.at[pl.ds(base + VBLK * t, VBLK)],
                    osems[slot]).wait()

            pltpu.async_copy(out_v.at[slot],
                             out_hbm.at[pl.ds(base + VBLK * t, VBLK)],
                             osems[slot])
        return carry

    nblk = VROWS_PER_W // VBLK
    lax.fori_loop(0, nblk // 2, block_step, 0)
    t = nblk - 1
    wait_in(0, t)

    def row_step_t(r5, c2):
        for rr in range(5):
            r = 5 * r5 + rr
            for c in range(EMBED_W // L):
                lo = in_v[0, r, pl.ds(L * c, L)]
                hi = in_v[0, r, pl.ds(EMBED_W + L * c, L)]
                out_v[0, r, pl.ds(L * c, L)] = _pack16(lo, hi)
        return c2

    lax.fori_loop(0, VBLK // 5, row_step_t, 0)
    pltpu.make_async_copy(out_v.at[0],
                          out_hbm.at[pl.ds(base + VBLK * (t - 2), VBLK)],
                          osems[0]).wait()
    pltpu.make_async_copy(out_v.at[1],
                          out_hbm.at[pl.ds(base + VBLK * (t - 1), VBLK)],
                          osems[1]).wait()
    pltpu.async_copy(out_v.at[0], out_hbm.at[pl.ds(base + VBLK * t, VBLK)],
                     osems[0])
    pltpu.make_async_copy(out_v.at[0],
                          out_hbm.at[pl.ds(base + VBLK * t, VBLK)],
                          osems[0]).wait()


_conv = pl.kernel(
    _conv_body,
    out_type=jax.ShapeDtypeStruct((VOCAB, EMBED_W), jnp.int32),
    mesh=plsc.VectorSubcoreMesh(core_axis_name="c", subcore_axis_name="s",
                                num_cores=NC, num_subcores=NS),
    compiler_params=pltpu.CompilerParams(use_tc_tiling_on_sc=False),
    scratch_types=[
        pltpu.VMEM((2, VBLK, EMBED), jnp.float32),
        pltpu.VMEM((2, VBLK, EMBED_W), jnp.int32),
        pltpu.SemaphoreType.DMA,
        pltpu.SemaphoreType.DMA,
        pltpu.SemaphoreType.DMA,
        pltpu.SemaphoreType.DMA,
    ],
)


def _pool_body(ids_hbm, table_hbm, out_hbm, idx_all, rows_v, acc_v,
               s00, s01, s10, s11, s20, s21, s30, s31,
               osem0, osem1, osem2, osem3):
    wid = lax.axis_index("s") * NC + lax.axis_index("c")
    base = wid * ROWS_PER_W
    sems = ((s00, s01), (s10, s11), (s20, s21), (s30, s31))
    osems = (osem0, osem1, osem2, osem3)

    pltpu.sync_copy(ids_hbm.at[pl.ds(base, ROWS_PER_W)], idx_all)

    def issue(slot, r):
        for j in range(2):
            pltpu.async_copy(table_hbm.at[idx_all.at[r, j]],
                             rows_v.at[slot, j], sems[slot][j])

    def wait(slot, r, j):
        pltpu.make_async_copy(table_hbm.at[idx_all.at[r, j]],
                              rows_v.at[slot, j], sems[slot][j]).wait()

    def chunk_reduce(slot, j, acc0):
        def seq_step(s, acc):
            acc = list(acc)
            for u in range(2):
                for k2 in range(NCHUNK // 2):
                    x32 = rows_v[slot, j, 2 * s + u, pl.ds(L * k2, L)]
                    a = lax.bitcast_convert_type(
                        lax.shift_left(x32, 16), jnp.float32)
                    bb = lax.bitcast_convert_type(
                        jnp.bitwise_and(x32, jnp.int32(-65536)),
                        jnp.float32)
                    acc[2 * k2] = acc[2 * k2] + a
                    acc[2 * k2 + 1] = acc[2 * k2 + 1] + bb
            return tuple(acc)

        return lax.fori_loop(0, SEQ_HALF // 2, seq_step, acc0)

    def reduce_store(slot, r, i):
        wait(slot, r, 0)
        acc = chunk_reduce(
            slot, 0, tuple(jnp.zeros((L,), jnp.float32)
                           for _ in range(NCHUNK)))
        wait(slot, r, 1)
        acc = chunk_reduce(slot, 1, acc)

        @pl.when(i > 0)
        def _():
            pltpu.make_async_copy(acc_v.at[slot], out_hbm.at[base + r],
                                  osems[slot]).wait()

        for k in range(NCHUNK):
            acc_v[slot, pl.ds(L * k, L)] = acc[k] * (1.0 / SEQ)
        pltpu.async_copy(acc_v.at[slot], out_hbm.at[base + r], osems[slot])

    issue(0, 0)
    issue(1, 1)

    def quad_step(i, carry):
        r0 = 4 * i
        issue(2, r0 + 2)
        issue(3, r0 + 3)
        reduce_store(0, r0, i)
        reduce_store(1, r0 + 1, i)

        @pl.when(r0 + 4 < ROWS_PER_W)
        def _():
            issue(0, r0 + 4)
            issue(1, r0 + 5)

        reduce_store(2, r0 + 2, i)
        reduce_store(3, r0 + 3, i)
        return carry

    lax.fori_loop(0, ROWS_PER_W // 4, quad_step, 0)

    last = ROWS_PER_W - 4
    for slot in range(4):
        pltpu.make_async_copy(acc_v.at[slot], out_hbm.at[base + last + slot],
                              osems[slot]).wait()


_pool = pl.kernel(
    _pool_body,
    out_type=jax.ShapeDtypeStruct((BATCH, EMBED), jnp.float32),
    mesh=plsc.VectorSubcoreMesh(core_axis_name="c", subcore_axis_name="s",
                                num_cores=NC, num_subcores=NS),
    compiler_params=pltpu.CompilerParams(use_tc_tiling_on_sc=False),
    scratch_types=[
        pltpu.VMEM((ROWS_PER_W, 2, SEQ_HALF), jnp.int32),
        pltpu.VMEM((4, 2, SEQ_HALF, EMBED_W), jnp.int32),
        pltpu.VMEM((4, EMBED), jnp.float32),
    ] + [pltpu.SemaphoreType.DMA] * 12,
)


def _linear_body(p_ref, w_ref, b_ref, o_ref):
    o_ref[...] = lax.dot_general(
        p_ref[...], w_ref[...], (((1,), (1,)), ((), ())),
        preferred_element_type=jnp.float32) + b_ref[...]


_PERM = np.concatenate(
    [np.concatenate([16 * k2 + np.arange(16),
                     64 + 16 * k2 + np.arange(16)])
     for k2 in range(4)]).astype(np.int32)


def kernel(input_ids, table, W, b):
    ids2 = input_ids.astype(jnp.int32).reshape(BATCH, 2, SEQ_HALF)
    packed = _conv(table)
    pooled = _pool(ids2, packed)
    out = pl.pallas_call(
        _linear_body,
        out_shape=jax.ShapeDtypeStruct((BATCH, HIDDEN), jnp.float32),
        grid=(BATCH // 1024,),
        in_specs=[
            pl.BlockSpec((1024, EMBED), lambda i: (i, 0)),
            pl.BlockSpec((HIDDEN, EMBED), lambda i: (0, 0)),
            pl.BlockSpec((1, HIDDEN), lambda i: (0, 0)),
        ],
        out_specs=pl.BlockSpec((1024, HIDDEN), lambda i: (i, 0)),
    )(pooled, W[:, _PERM], b.reshape(1, HIDDEN))
    return out
